# baseline (speedup 1.0000x reference)
import functools

import jax
import jax.numpy as jnp
from jax import lax
from jax.experimental import pallas as pl
from jax.experimental.pallas import tpu as pltpu
from jax.experimental.pallas import tpu_sc as plsc

BATCH = 16384
EMBED_DIM = 32
NUM_CHUNKS = 4
CHUNK = 128
PER_WORKER = NUM_CHUNKS * CHUNK


def _sc_body(ue_hbm, uo_hbm, ie_hbm, io_hbm, utab_hbm, itab_hbm, out_hbm,
             ue_i, uo_i, ie_i, io_i, ue_v, uo_v, ie_v, io_v, out_v,
             sem_u, sem_v):
    nc = 2
    wid = lax.axis_index("s") * nc + lax.axis_index("c")

    pltpu.sync_copy(ue_hbm.at[wid], ue_i)
    pltpu.sync_copy(uo_hbm.at[wid], uo_i)
    pltpu.sync_copy(ie_hbm.at[wid], ie_i)
    pltpu.sync_copy(io_hbm.at[wid], io_i)

    copies = []
    for j in range(NUM_CHUNKS):
        copies.append(
            pltpu.async_copy(utab_hbm.at[ue_i.at[j]], ue_v.at[j], sem_u))
        copies.append(
            pltpu.async_copy(utab_hbm.at[uo_i.at[j]], uo_v.at[j], sem_u))
        copies.append(
            pltpu.async_copy(itab_hbm.at[ie_i.at[j]], ie_v.at[j], sem_v))
        copies.append(
            pltpu.async_copy(itab_hbm.at[io_i.at[j]], io_v.at[j], sem_v))
    for c in copies:
        c.wait()

    lanes = lax.iota(jnp.int32, 16)
    perms = [jnp.bitwise_xor(lanes, k) for k in (8, 4, 2, 1)]

    def group_body(j, g):
        def row_body(r, acc):
            row = g * 16 + r
            ue = ue_v[j, row, pl.ds(0, 16)]
            uo = uo_v[j, row, pl.ds(0, 16)]
            ve = ie_v[j, row, pl.ds(0, 16)]
            vo = io_v[j, row, pl.ds(0, 16)]
            t = ue * ve + uo * vo
            for p in perms:
                t = t + t.at[p].get(mode="promise_in_bounds",
                                    unique_indices=True)
            return jnp.where(lanes == r, t, acc)

        acc = lax.fori_loop(0, 16, row_body, jnp.zeros((16,), jnp.float32))
        out_v[j, pl.ds(g * 16, 16)] = acc

    for j in range(NUM_CHUNKS):
        lax.fori_loop(0, CHUNK // 16,
                      lambda g, _, j=j: (group_body(j, g), 0)[1], 0)

    pltpu.sync_copy(out_v, out_hbm.at[wid])


@jax.jit
def _mf_dot(ue, uo, ie, io, utab2, itab2):
    mesh = plsc.VectorSubcoreMesh(core_axis_name="c", subcore_axis_name="s")
    f = functools.partial(
        pl.kernel,
        mesh=mesh,
        compiler_params=pltpu.CompilerParams(use_tc_tiling_on_sc=False),
        out_type=jax.ShapeDtypeStruct((32, NUM_CHUNKS, CHUNK), jnp.float32),
        scratch_types=[
            pltpu.VMEM((NUM_CHUNKS, CHUNK), jnp.int32),
            pltpu.VMEM((NUM_CHUNKS, CHUNK), jnp.int32),
            pltpu.VMEM((NUM_CHUNKS, CHUNK), jnp.int32),
            pltpu.VMEM((NUM_CHUNKS, CHUNK), jnp.int32),
            pltpu.VMEM((NUM_CHUNKS, CHUNK, 16), jnp.float32),
            pltpu.VMEM((NUM_CHUNKS, CHUNK, 16), jnp.float32),
            pltpu.VMEM((NUM_CHUNKS, CHUNK, 16), jnp.float32),
            pltpu.VMEM((NUM_CHUNKS, CHUNK, 16), jnp.float32),
            pltpu.VMEM((NUM_CHUNKS, CHUNK), jnp.float32),
            pltpu.SemaphoreType.DMA,
            pltpu.SemaphoreType.DMA,
        ],
    )(_sc_body)
    return f(ue, uo, ie, io, utab2, itab2)


def kernel(X, user_table, item_table):
    uid2 = (X[:, 0] * 2).reshape(32, NUM_CHUNKS, CHUNK)
    iid2 = (X[:, 1] * 2).reshape(32, NUM_CHUNKS, CHUNK)
    utab2 = user_table.reshape(-1, 16)
    itab2 = item_table.reshape(-1, 16)
    out = _mf_dot(uid2, uid2 + 1, iid2, iid2 + 1, utab2, itab2)
    return out.reshape(BATCH, 1)
